# Initial kernel scaffold; baseline (speedup 1.0000x reference)
import jax
import jax.numpy as jnp
from jax.experimental import pallas as pl


def kernel(x, table):
    raise NotImplementedError("write your pallas kernel here")


# speedup vs baseline: 1.1020x; 1.1020x over previous
import functools

import jax
import jax.numpy as jnp
from jax import lax
from jax.experimental import pallas as pl
from jax.experimental.pallas import tpu as pltpu
from jax.experimental.pallas import tpu_sc as plsc

_NC = 2
_NS = 16
_NW = _NC * _NS

_CHUNK = 1600


def _emb_gather(table, idx):
  total = idx.shape[0]
  b_per_w = total // _NW
  nchunk = b_per_w // _CHUNK
  emb_dim = table.shape[1]
  mesh = plsc.VectorSubcoreMesh(core_axis_name="c", subcore_axis_name="s")

  @functools.partial(
      pl.kernel,
      out_type=jax.ShapeDtypeStruct((total, emb_dim), jnp.float32),
      mesh=mesh,
      scratch_types=[
          pltpu.VMEM((_CHUNK,), jnp.int32),
          pltpu.VMEM((_CHUNK, emb_dim), jnp.float32),
          pltpu.SemaphoreType.DMA,
      ],
      compiler_params=pltpu.CompilerParams(use_tc_tiling_on_sc=False),
  )
  def k(table_hbm, idx_hbm, out_hbm, idx_v, rows_v, sem):
    wid = lax.axis_index("s") * _NC + lax.axis_index("c")
    base = wid * b_per_w
    for c in range(nchunk):
      off = base + c * _CHUNK
      pltpu.sync_copy(idx_hbm.at[pl.ds(off, _CHUNK)], idx_v)
      pltpu.async_copy(table_hbm.at[idx_v], rows_v, sem).wait()
      pltpu.sync_copy(rows_v, out_hbm.at[pl.ds(off, _CHUNK)])

  return k(table, idx)


def kernel(x, table):
  idx = x.reshape(-1).astype(jnp.int32)
  out = _emb_gather(table, idx)
  return out.reshape(x.shape + (table.shape[1],))


# speedup vs baseline: 1.1085x; 1.0059x over previous
import functools

import jax
import jax.numpy as jnp
from jax import lax
from jax.experimental import pallas as pl
from jax.experimental.pallas import tpu as pltpu
from jax.experimental.pallas import tpu_sc as plsc

_NC = 2
_NS = 16
_NW = _NC * _NS

_CHUNK = 1600
_NBUF = 2


def _emb_gather(table, idx):
  total = idx.shape[0]
  b_per_w = total // _NW
  nchunk = b_per_w // _CHUNK
  emb_dim = table.shape[1]
  mesh = plsc.VectorSubcoreMesh(core_axis_name="c", subcore_axis_name="s")

  scratch = (
      [pltpu.VMEM((_CHUNK,), jnp.int32) for _ in range(_NBUF)]
      + [pltpu.VMEM((_CHUNK, emb_dim), jnp.float32) for _ in range(_NBUF)]
      + [pltpu.SemaphoreType.DMA for _ in range(3 * _NBUF)]
  )

  @functools.partial(
      pl.kernel,
      out_type=jax.ShapeDtypeStruct((total, emb_dim), jnp.float32),
      mesh=mesh,
      scratch_types=scratch,
      compiler_params=pltpu.CompilerParams(use_tc_tiling_on_sc=False),
  )
  def k(table_hbm, idx_hbm, out_hbm, *refs):
    idx_bufs = refs[0:_NBUF]
    row_bufs = refs[_NBUF:2 * _NBUF]
    sem_i = refs[2 * _NBUF:2 * _NBUF + _NBUF]
    sem_g = refs[3 * _NBUF:3 * _NBUF + _NBUF]
    sem_o = refs[4 * _NBUF:4 * _NBUF + _NBUF]

    wid = lax.axis_index("s") * _NC + lax.axis_index("c")
    base = wid * b_per_w

    def idx_copy(c):
      b = c % _NBUF
      return pltpu.async_copy(
          idx_hbm.at[pl.ds(base + c * _CHUNK, _CHUNK)], idx_bufs[b], sem_i[b])

    def gather(c):
      b = c % _NBUF
      return pltpu.async_copy(table_hbm.at[idx_bufs[b]], row_bufs[b], sem_g[b])

    def out_copy(c):
      b = c % _NBUF
      return pltpu.async_copy(
          row_bufs[b], out_hbm.at[pl.ds(base + c * _CHUNK, _CHUNK)], sem_o[b])

    cp_i, cp_g, cp_o = {}, {}, {}
    for t in range(nchunk + 2):
      c = t - 2
      if 0 <= c < nchunk:
        cp_g[c].wait()
        cp_o[c] = out_copy(c)
      c = t - 1
      if 0 <= c < nchunk:
        cp_i[c].wait()
        if c - _NBUF >= 0:
          cp_o.pop(c - _NBUF).wait()
        cp_g[c] = gather(c)
      if t < nchunk:
        cp_i[t] = idx_copy(t)
    for c in sorted(cp_o):
      cp_o[c].wait()

  return k(table, idx)


def kernel(x, table):
  idx = x.reshape(-1).astype(jnp.int32)
  out = _emb_gather(table, idx)
  return out.reshape(x.shape + (table.shape[1],))


# speedup vs baseline: 1.7858x; 1.6110x over previous
import functools

import jax
import jax.numpy as jnp
from jax import lax
from jax.experimental import pallas as pl
from jax.experimental.pallas import tpu as pltpu
from jax.experimental.pallas import tpu_sc as plsc

_NC = 2
_NS = 16
_NW = _NC * _NS

_CHUNK = 1600
_NBUF = 2


def _emb_gather(table, idx):
  total = idx.shape[0]
  b_per_w = total // _NW
  nchunk = b_per_w // _CHUNK
  emb_dim = table.shape[1]
  mesh = plsc.VectorSubcoreMesh(core_axis_name="c", subcore_axis_name="s")

  scratch = (
      [pltpu.VMEM((_CHUNK,), jnp.int32) for _ in range(_NBUF)]
      + [pltpu.VMEM((_CHUNK, emb_dim), jnp.float32) for _ in range(_NBUF)]
      + [pltpu.SemaphoreType.DMA for _ in range(3 * _NBUF)]
  )

  @functools.partial(
      pl.kernel,
      out_type=jax.ShapeDtypeStruct((total // 50, 50, emb_dim), jnp.float32),
      mesh=mesh,
      scratch_types=scratch,
      compiler_params=pltpu.CompilerParams(use_tc_tiling_on_sc=False),
  )
  def k(table_hbm, idx_hbm, out_3d, *refs):
    idx_bufs = refs[0:_NBUF]
    row_bufs = refs[_NBUF:2 * _NBUF]
    sem_i = refs[2 * _NBUF:2 * _NBUF + _NBUF]
    sem_g = refs[3 * _NBUF:3 * _NBUF + _NBUF]
    sem_o = refs[4 * _NBUF:4 * _NBUF + _NBUF]

    wid = lax.axis_index("s") * _NC + lax.axis_index("c")
    base = wid * b_per_w

    def idx_copy(c):
      b = c % _NBUF
      return pltpu.async_copy(
          idx_hbm.at[pl.ds(base + c * _CHUNK, _CHUNK)], idx_bufs[b], sem_i[b])

    def gather(c):
      b = c % _NBUF
      return pltpu.async_copy(table_hbm.at[idx_bufs[b]], row_bufs[b], sem_g[b])

    batches_per_chunk = _CHUNK // 50

    def out_copy(c):
      b = c % _NBUF
      b0 = (base + c * _CHUNK) // 50
      return [
          pltpu.async_copy(
              row_bufs[b].at[pl.ds(j * 50, 50)], out_3d.at[b0 + j], sem_o[b])
          for j in range(batches_per_chunk)
      ]

    cp_i, cp_g, cp_o = {}, {}, {}
    for t in range(nchunk + 2):
      c = t - 2
      if 0 <= c < nchunk:
        cp_g[c].wait()
        cp_o[c] = out_copy(c)
      c = t - 1
      if 0 <= c < nchunk:
        cp_i[c].wait()
        if c - _NBUF >= 0:
          for d in cp_o.pop(c - _NBUF):
            d.wait()
        cp_g[c] = gather(c)
      if t < nchunk:
        cp_i[t] = idx_copy(t)
    for c in sorted(cp_o):
      for d in cp_o[c]:
        d.wait()

  return k(table, idx)


def kernel(x, table):
  idx = x.reshape(-1).astype(jnp.int32)
  return _emb_gather(table, idx)
